# baseline (speedup 1.0000x reference)
import functools

import jax
import jax.numpy as jnp
from jax import lax
from jax.experimental import pallas as pl
from jax.experimental.pallas import tpu as pltpu
from jax.experimental.pallas import tpu_sc as plsc

_MARGIN = 3.0
_DIM = 64
_LANES = 16
_CHUNK = 128


def _make_sc_kernel(batch, num_ent, num_rel):
    info = plsc.get_sparse_core_info()
    nw = info.num_cores * info.num_subcores
    per_w = batch // nw
    n_chunks = per_w // _CHUNK
    mesh = plsc.VectorSubcoreMesh(core_axis_name="c", subcore_axis_name="s")

    @functools.partial(
        pl.kernel,
        mesh=mesh,
        out_type=jax.ShapeDtypeStruct((nw, _LANES), jnp.float32),
        compiler_params=pltpu.CompilerParams(
            use_tc_tiling_on_sc=False, needs_layout_passes=False),
        scratch_types=[
            pltpu.VMEM((_CHUNK,), jnp.int32),
            pltpu.VMEM((_CHUNK,), jnp.int32),
            pltpu.VMEM((_CHUNK,), jnp.int32),
            pltpu.VMEM((_CHUNK,), jnp.int32),
            pltpu.VMEM((_CHUNK,), jnp.int32),
            pltpu.VMEM((_CHUNK, _DIM), jnp.float32),
            pltpu.VMEM((_CHUNK, _DIM), jnp.float32),
            pltpu.VMEM((_CHUNK, _DIM), jnp.float32),
            pltpu.VMEM((_CHUNK, _DIM), jnp.float32),
            pltpu.VMEM((_CHUNK, _DIM), jnp.float32),
            pltpu.VMEM((_LANES,), jnp.float32),
            pltpu.SemaphoreType.DMA,
        ],
    )
    def trans_e(ph_hbm, pr_hbm, pt_hbm, nh_hbm, nt_hbm, ent_hbm, rel_hbm,
                out_hbm, ph_i, pr_i, pt_i, nh_i, nt_i,
                h_rows, r_rows, t_rows, nh_rows, nt_rows, part_v, sem):
        wid = lax.axis_index("s") * info.num_cores + lax.axis_index("c")
        lane = lax.iota(jnp.int32, _LANES)
        zero16 = jnp.zeros((_LANES,), jnp.float32)

        def chunk_body(c, part):
            base = wid * per_w + c * _CHUNK
            pltpu.sync_copy(ph_hbm.at[pl.ds(base, _CHUNK)], ph_i)
            pltpu.sync_copy(pr_hbm.at[pl.ds(base, _CHUNK)], pr_i)
            pltpu.sync_copy(pt_hbm.at[pl.ds(base, _CHUNK)], pt_i)
            pltpu.sync_copy(nh_hbm.at[pl.ds(base, _CHUNK)], nh_i)
            pltpu.sync_copy(nt_hbm.at[pl.ds(base, _CHUNK)], nt_i)
            d1 = pltpu.async_copy(ent_hbm.at[ph_i], h_rows, sem)
            d2 = pltpu.async_copy(rel_hbm.at[pr_i], r_rows, sem)
            d3 = pltpu.async_copy(ent_hbm.at[pt_i], t_rows, sem)
            d4 = pltpu.async_copy(ent_hbm.at[nh_i], nh_rows, sem)
            d5 = pltpu.async_copy(ent_hbm.at[nt_i], nt_rows, sem)
            d1.wait()
            d2.wait()
            d3.wait()
            d4.wait()
            d5.wait()

            def g_body(g, part):
                row = lane + g * _LANES

                def k_body(k, acc):
                    for u in range(4):
                        col = jnp.zeros((_LANES,), jnp.int32) + (k * 4 + u)
                        hv = plsc.load_gather(h_rows, [row, col])
                        rv = plsc.load_gather(r_rows, [row, col])
                        tv = plsc.load_gather(t_rows, [row, col])
                        nhv = plsc.load_gather(nh_rows, [row, col])
                        ntv = plsc.load_gather(nt_rows, [row, col])
                        acc = acc + (jnp.abs(hv + rv - tv)
                                     - jnp.abs(nhv + rv - ntv))
                    return acc

                diff = lax.fori_loop(0, _DIM // 4, k_body, zero16)
                return part + jnp.maximum(diff + _MARGIN, 0.0)

            return lax.fori_loop(0, _CHUNK // _LANES, g_body, part)

        part = lax.fori_loop(0, n_chunks, chunk_body, zero16)
        part_v[...] = part
        pltpu.sync_copy(part_v, out_hbm.at[wid])

    return trans_e


@jax.jit
def kernel(pos_h, pos_r, pos_t, neg_h, neg_t, ent_emb, rel_emb):
    batch = pos_h.shape[0]
    sc_fn = _make_sc_kernel(batch, ent_emb.shape[0], rel_emb.shape[0])
    partials = sc_fn(pos_h.astype(jnp.int32), pos_r.astype(jnp.int32),
                     pos_t.astype(jnp.int32), neg_h.astype(jnp.int32),
                     neg_t.astype(jnp.int32), ent_emb, rel_emb)
    return jnp.sum(partials) / batch


# speedup vs baseline: 1.8377x; 1.8377x over previous
import functools

import jax
import jax.numpy as jnp
from jax import lax
from jax.experimental import pallas as pl
from jax.experimental.pallas import tpu as pltpu
from jax.experimental.pallas import tpu_sc as plsc

_MARGIN = 3.0
_DIM = 64
_LANES = 16
_CHUNK = 128


def _make_sc_kernel(batch):
    info = plsc.get_sparse_core_info()
    nw = info.num_cores * info.num_subcores
    per_w = batch // nw
    n_chunks = per_w // _CHUNK
    mesh = plsc.VectorSubcoreMesh(core_axis_name="c", subcore_axis_name="s")

    @functools.partial(
        pl.kernel,
        mesh=mesh,
        out_type=jax.ShapeDtypeStruct((nw, _LANES), jnp.float32),
        compiler_params=pltpu.CompilerParams(needs_layout_passes=False),
        scratch_types=[
            pltpu.VMEM((_CHUNK,), jnp.int32),
            pltpu.VMEM((_CHUNK,), jnp.int32),
            pltpu.VMEM((_CHUNK,), jnp.int32),
            pltpu.VMEM((_CHUNK,), jnp.int32),
            pltpu.VMEM((_CHUNK,), jnp.int32),
            pltpu.VMEM((_CHUNK, _DIM), jnp.float32),
            pltpu.VMEM((_CHUNK, _DIM), jnp.float32),
            pltpu.VMEM((_CHUNK, _DIM), jnp.float32),
            pltpu.VMEM((_CHUNK, _DIM), jnp.float32),
            pltpu.VMEM((_CHUNK, _DIM), jnp.float32),
            pltpu.VMEM((_LANES,), jnp.float32),
            pltpu.SemaphoreType.DMA,
        ],
    )
    def trans_e(ph_hbm, pr_hbm, pt_hbm, nh_hbm, nt_hbm, ent_hbm, rel_hbm,
                out_hbm, ph_i, pr_i, pt_i, nh_i, nt_i,
                h_rows, r_rows, t_rows, nh_rows, nt_rows, part_v, sem):
        wid = lax.axis_index("s") * info.num_cores + lax.axis_index("c")
        zero16 = jnp.zeros((_LANES,), jnp.float32)

        def chunk_body(c, part):
            base = wid * per_w + c * _CHUNK
            pltpu.sync_copy(ph_hbm.at[pl.ds(base, _CHUNK)], ph_i)
            pltpu.sync_copy(pr_hbm.at[pl.ds(base, _CHUNK)], pr_i)
            pltpu.sync_copy(pt_hbm.at[pl.ds(base, _CHUNK)], pt_i)
            pltpu.sync_copy(nh_hbm.at[pl.ds(base, _CHUNK)], nh_i)
            pltpu.sync_copy(nt_hbm.at[pl.ds(base, _CHUNK)], nt_i)

            def fire_body(g, carry):
                base16 = pl.ds(g * _LANES, _LANES)
                phv, prv = ph_i[base16], pr_i[base16]
                ptv, nhv, ntv = pt_i[base16], nh_i[base16], nt_i[base16]
                for u in range(_LANES):
                    j = g * _LANES + u
                    pltpu.async_copy(ent_hbm.at[phv[u]], h_rows.at[j], sem)
                    pltpu.async_copy(rel_hbm.at[prv[u]], r_rows.at[j], sem)
                    pltpu.async_copy(ent_hbm.at[ptv[u]], t_rows.at[j], sem)
                    pltpu.async_copy(ent_hbm.at[nhv[u]], nh_rows.at[j], sem)
                    pltpu.async_copy(ent_hbm.at[ntv[u]], nt_rows.at[j], sem)
                return carry

            lax.fori_loop(0, _CHUNK // _LANES, fire_body, 0)
            for buf in (h_rows, r_rows, t_rows, nh_rows, nt_rows):
                pltpu.make_async_copy(
                    ent_hbm.at[pl.ds(0, _CHUNK)], buf, sem).wait()

            def elem_body(j, acc):
                d = zero16
                for u in range(_DIM // _LANES):
                    s = pl.ds(u * _LANES, _LANES)
                    d = d + (jnp.abs(h_rows[j, s] + r_rows[j, s]
                                     - t_rows[j, s])
                             - jnp.abs(nh_rows[j, s] + r_rows[j, s]
                                       - nt_rows[j, s]))
                return acc + jnp.maximum(jnp.sum(d) + _MARGIN, 0.0)

            return lax.fori_loop(0, _CHUNK, elem_body, part)

        part = lax.fori_loop(0, n_chunks, chunk_body, jnp.float32(0.0))
        part_v[...] = zero16 + part * (1.0 / 16.0)
        pltpu.sync_copy(part_v, out_hbm.at[wid])

    return trans_e


@jax.jit
def kernel(pos_h, pos_r, pos_t, neg_h, neg_t, ent_emb, rel_emb):
    batch = pos_h.shape[0]
    sc_fn = _make_sc_kernel(batch)
    partials = sc_fn(pos_h.astype(jnp.int32), pos_r.astype(jnp.int32),
                     pos_t.astype(jnp.int32), neg_h.astype(jnp.int32),
                     neg_t.astype(jnp.int32), ent_emb, rel_emb)
    return jnp.sum(partials) / batch
